# Initial kernel scaffold; baseline (speedup 1.0000x reference)
import jax
import jax.numpy as jnp
from jax.experimental import pallas as pl


def kernel(x, c1w, bn1g, bn1b, c2w, bn2g, bn2b, c3w, bn3g, bn3b, c4w, bn4g, bn4b, c5w, bn5g, bn5b, c6w, bn6g, bn6b, g1w, g1b, gbng, gbnb, g2w, g2b):
    raise NotImplementedError("write your pallas kernel here")


# speedup vs baseline: 7.2934x; 7.2934x over previous
import functools

import jax
import jax.numpy as jnp
from jax import lax
from jax.experimental import pallas as pl
from jax.experimental.pallas import tpu as pltpu
from jax.experimental.pallas import tpu_sc as plsc

EPS = 1e-5
KNB = 20
NBLK = 256


def _knn_body(xb_ref, xf_ref, idx_ref):
    b = pl.program_id(0)
    xb = xb_ref[0]
    xf = xf_ref[0]
    n = xf.shape[0]
    nb = xb.shape[0]
    inner = -2.0 * lax.dot_general(
        xf, xb, (((1,), (1,)), ((), ())), preferred_element_type=jnp.float32)
    sqf = jnp.sum(xf * xf, axis=1, keepdims=True)
    ones = jnp.ones((1, xb.shape[1]), jnp.float32)
    sqb = lax.dot_general(
        ones, xb * xb, (((1,), (1,)), ((), ())),
        preferred_element_type=jnp.float32)
    pd = (-sqf) - inner - sqb
    iot = lax.broadcasted_iota(jnp.int32, (n, nb), 0)
    base = b * n
    neg = jnp.float32(-jnp.inf)
    for t in range(KNB):
        m = jnp.max(pd, axis=0, keepdims=True)
        am = jnp.min(jnp.where(pd == m, iot, n), axis=0)
        idx_ref[0, t, :] = am + base
        pd = jnp.where(iot == am[None, :], neg, pd)


def _knn(x):
    B, N, C = x.shape
    return pl.pallas_call(
        _knn_body,
        grid=(B, N // NBLK),
        in_specs=[
            pl.BlockSpec((1, NBLK, C), lambda b, nb: (b, nb, 0)),
            pl.BlockSpec((1, N, C), lambda b, nb: (b, 0, 0)),
        ],
        out_specs=pl.BlockSpec((1, KNB, NBLK), lambda b, nb: (b, 0, nb)),
        out_shape=jax.ShapeDtypeStruct((B, KNB, N), jnp.int32),
    )(x, x)


def _gather_rows(table, idxflat):
    R = idxflat.shape[0]
    D = table.shape[1]
    NC, NS = 2, 16
    NW = NC * NS
    per_w = R // NW
    CH = 1024
    n_ch = per_w // CH
    mesh = plsc.VectorSubcoreMesh(core_axis_name="c", subcore_axis_name="s")

    @functools.partial(
        pl.kernel,
        mesh=mesh,
        compiler_params=pltpu.CompilerParams(use_tc_tiling_on_sc=False),
        out_type=jax.ShapeDtypeStruct((R, D), jnp.float32),
        scratch_types=[
            pltpu.VMEM((CH,), jnp.int32),
            pltpu.VMEM((CH, D), jnp.float32),
            pltpu.SemaphoreType.DMA,
        ],
    )
    def k(table_hbm, idx_hbm, out_hbm, idx_v, rows_v, sem):
        wid = lax.axis_index("s") * NC + lax.axis_index("c")
        base = wid * per_w
        for c in range(n_ch):
            off = base + c * CH
            pltpu.sync_copy(idx_hbm.at[pl.ds(off, CH)], idx_v)
            pltpu.async_copy(table_hbm.at[idx_v], rows_v, sem).wait()
            pltpu.sync_copy(rows_v, out_hbm.at[pl.ds(off, CH)])

    return k(table, idxflat)


def _acc_stats(part, o_ref, acc_ref):
    b = pl.program_id(0)
    nb = pl.program_id(1)
    first = jnp.logical_and(b == 0, nb == 0)

    @pl.when(first)
    def _init():
        acc_ref[...] = part

    @pl.when(jnp.logical_not(first))
    def _acc():
        acc_ref[...] = acc_ref[...] + part

    @pl.when(jnp.logical_and(b == pl.num_programs(0) - 1,
                             nb == pl.num_programs(1) - 1))
    def _fin():
        o_ref[...] = acc_ref[...]


def _p1_body(gx_ref, x_ref, w_ref, y_ref, o_ref, acc_ref, *, c_real):
    gx = gx_ref[0][:, :, :c_real]
    xi = x_ref[0][:, :c_real]
    K, Nb, C = gx.shape
    xib = jnp.broadcast_to(xi[None], (K, Nb, C))
    f = jnp.concatenate([gx - xib, xib], axis=2)
    y = jnp.dot(f.reshape(K * Nb, 2 * C), w_ref[...],
                preferred_element_type=jnp.float32)
    DO = y.shape[1]
    y3 = y.reshape(K, Nb, DO)
    y_ref[0] = y3
    s = jnp.sum(jnp.sum(y3, axis=0), axis=0)
    ss = jnp.sum(jnp.sum(y3 * y3, axis=0), axis=0)
    _acc_stats(jnp.concatenate([s[None, :], ss[None, :]], axis=0),
               o_ref, acc_ref)


def _edge_conv1(Gx, x, wT):
    B, K, N, Cp = Gx.shape
    DO = wT.shape[1]
    c_real = wT.shape[0] // 2
    return pl.pallas_call(
        functools.partial(_p1_body, c_real=c_real),
        grid=(B, N // NBLK),
        in_specs=[
            pl.BlockSpec((1, K, NBLK, Cp), lambda b, nb: (b, 0, nb, 0)),
            pl.BlockSpec((1, NBLK, Cp), lambda b, nb: (b, nb, 0)),
            pl.BlockSpec((2 * c_real, DO), lambda b, nb: (0, 0)),
        ],
        out_specs=[
            pl.BlockSpec((1, K, NBLK, DO), lambda b, nb: (b, 0, nb, 0)),
            pl.BlockSpec((2, DO), lambda b, nb: (0, 0)),
        ],
        out_shape=[
            jax.ShapeDtypeStruct((B, K, N, DO), jnp.float32),
            jax.ShapeDtypeStruct((2, DO), jnp.float32),
        ],
        scratch_shapes=[pltpu.VMEM((2, DO), jnp.float32)],
    )(Gx, x, wT)


def _bn_coeffs(st, cnt):
    mean = st[0:1, :] / cnt
    var = st[1:2, :] / cnt - mean * mean
    return mean, jnp.sqrt(var + EPS)


def _xla_stats(Y):
    Yt = jnp.transpose(Y, (0, 3, 2, 1))
    m = jnp.mean(Yt, axis=(0, 2, 3))
    v = jnp.var(Yt, axis=(0, 2, 3))
    return m.reshape(1, -1), jnp.sqrt(v + EPS).reshape(1, -1)


def _bn_apply(y, mean, s, gam, bet):
    return (y - mean) / s * gam + bet


def _p2_body(y_ref, mn_ref, s_ref, gam_ref, bet_ref, w_ref, z_ref, o_ref,
             acc_ref):
    y = y_ref[0]
    yn = _bn_apply(y, mn_ref[...][None], s_ref[...][None],
                   gam_ref[...][None], bet_ref[...][None])
    yn = jnp.where(yn >= 0, yn, 0.2 * yn)
    K, Nb, D = yn.shape
    z = jnp.dot(yn.reshape(K * Nb, D), w_ref[...],
                preferred_element_type=jnp.float32)
    z3 = z.reshape(K, Nb, -1)
    z_ref[0] = z3
    s = jnp.sum(jnp.sum(z3, axis=0), axis=0)
    ss = jnp.sum(jnp.sum(z3 * z3, axis=0), axis=0)
    _acc_stats(jnp.concatenate([s[None, :], ss[None, :]], axis=0),
               o_ref, acc_ref)


def _bn_lrelu_conv(Y, mn, sd, gam, bet, wT):
    B, K, N, D = Y.shape
    DO = wT.shape[1]
    return pl.pallas_call(
        _p2_body,
        grid=(B, N // NBLK),
        in_specs=[
            pl.BlockSpec((1, K, NBLK, D), lambda b, nb: (b, 0, nb, 0)),
            pl.BlockSpec((1, D), lambda b, nb: (0, 0)),
            pl.BlockSpec((1, D), lambda b, nb: (0, 0)),
            pl.BlockSpec((1, D), lambda b, nb: (0, 0)),
            pl.BlockSpec((1, D), lambda b, nb: (0, 0)),
            pl.BlockSpec((D, DO), lambda b, nb: (0, 0)),
        ],
        out_specs=[
            pl.BlockSpec((1, K, NBLK, DO), lambda b, nb: (b, 0, nb, 0)),
            pl.BlockSpec((2, DO), lambda b, nb: (0, 0)),
        ],
        out_shape=[
            jax.ShapeDtypeStruct((B, K, N, DO), jnp.float32),
            jax.ShapeDtypeStruct((2, DO), jnp.float32),
        ],
        scratch_shapes=[pltpu.VMEM((2, DO), jnp.float32)],
    )(Y, mn, sd, gam, bet, wT)


def _p3_body(z_ref, mn_ref, s_ref, gam_ref, bet_ref, x_ref):
    zn = _bn_apply(z_ref[0], mn_ref[...][None], s_ref[...][None],
                   gam_ref[...][None], bet_ref[...][None])
    zn = jnp.where(zn >= 0, zn, 0.2 * zn)
    x_ref[0] = jnp.max(zn, axis=0)


def _bn_lrelu_max(Z, mn, sd, gam, bet):
    B, K, N, D = Z.shape
    return pl.pallas_call(
        _p3_body,
        grid=(B, N // NBLK),
        in_specs=[
            pl.BlockSpec((1, K, NBLK, D), lambda b, nb: (b, 0, nb, 0)),
            pl.BlockSpec((1, D), lambda b, nb: (0, 0)),
            pl.BlockSpec((1, D), lambda b, nb: (0, 0)),
            pl.BlockSpec((1, D), lambda b, nb: (0, 0)),
            pl.BlockSpec((1, D), lambda b, nb: (0, 0)),
        ],
        out_specs=pl.BlockSpec((1, NBLK, D), lambda b, nb: (b, nb, 0)),
        out_shape=jax.ShapeDtypeStruct((B, N, D), jnp.float32),
    )(Z, mn, sd, gam, bet)


def _c6_body(x1_ref, x2_ref, x3_ref, w_ref, y_ref, o_ref, acc_ref):
    hc = jnp.concatenate([x1_ref[0], x2_ref[0], x3_ref[0]], axis=1)
    y = jnp.dot(hc, w_ref[...], preferred_element_type=jnp.float32)
    y_ref[0] = y
    s = jnp.sum(y, axis=0)
    ss = jnp.sum(y * y, axis=0)
    _acc_stats(jnp.concatenate([s[None, :], ss[None, :]], axis=0),
               o_ref, acc_ref)


def _conv6(x1, x2, x3, w6T):
    B, N, D = x1.shape
    DO = w6T.shape[1]
    return pl.pallas_call(
        _c6_body,
        grid=(B, N // NBLK),
        in_specs=[
            pl.BlockSpec((1, NBLK, D), lambda b, nb: (b, nb, 0)),
            pl.BlockSpec((1, NBLK, D), lambda b, nb: (b, nb, 0)),
            pl.BlockSpec((1, NBLK, D), lambda b, nb: (b, nb, 0)),
            pl.BlockSpec((3 * D, DO), lambda b, nb: (0, 0)),
        ],
        out_specs=[
            pl.BlockSpec((1, NBLK, DO), lambda b, nb: (b, nb, 0)),
            pl.BlockSpec((2, DO), lambda b, nb: (0, 0)),
        ],
        out_shape=[
            jax.ShapeDtypeStruct((B, N, DO), jnp.float32),
            jax.ShapeDtypeStruct((2, DO), jnp.float32),
        ],
        scratch_shapes=[pltpu.VMEM((2, DO), jnp.float32)],
    )(x1, x2, x3, w6T)


def _q_body(y_ref, mn_ref, s_ref, gam_ref, bet_ref, loc_ref, gx_ref, mx_ref):
    nb = pl.program_id(1)
    ln = _bn_apply(y_ref[0], mn_ref[...], s_ref[...], gam_ref[...], bet_ref[...])
    ln = jnp.where(ln >= 0, ln, 0.2 * ln)
    loc_ref[0] = ln
    m = jnp.max(ln, axis=0, keepdims=True)

    @pl.when(nb == 0)
    def _init():
        mx_ref[...] = m

    @pl.when(nb > 0)
    def _acc():
        mx_ref[...] = jnp.maximum(mx_ref[...], m)

    @pl.when(nb == pl.num_programs(1) - 1)
    def _fin():
        gx_ref[0] = mx_ref[...]


def _bn_lrelu_gmax(Y, mn, sd, gam, bet):
    B, N, DO = Y.shape
    return pl.pallas_call(
        _q_body,
        grid=(B, N // NBLK),
        in_specs=[
            pl.BlockSpec((1, NBLK, DO), lambda b, nb: (b, nb, 0)),
            pl.BlockSpec((1, DO), lambda b, nb: (0, 0)),
            pl.BlockSpec((1, DO), lambda b, nb: (0, 0)),
            pl.BlockSpec((1, DO), lambda b, nb: (0, 0)),
            pl.BlockSpec((1, DO), lambda b, nb: (0, 0)),
        ],
        out_specs=[
            pl.BlockSpec((1, NBLK, DO), lambda b, nb: (b, nb, 0)),
            pl.BlockSpec((1, 1, DO), lambda b, nb: (b, 0, 0)),
        ],
        out_shape=[
            jax.ShapeDtypeStruct((B, N, DO), jnp.float32),
            jax.ShapeDtypeStruct((B, 1, DO), jnp.float32),
        ],
        scratch_shapes=[pltpu.VMEM((1, DO), jnp.float32)],
    )(Y, mn, sd, gam, bet)


def _head_body(gx_ref, w1_ref, b1_ref, gam_ref, bet_ref, w2_ref, b2_ref,
               out_ref):
    a = jnp.dot(gx_ref[...], w1_ref[...],
                preferred_element_type=jnp.float32) + b1_ref[...]
    m = jnp.mean(a, axis=0, keepdims=True)
    v = jnp.mean((a - m) * (a - m), axis=0, keepdims=True)
    an = (a - m) / jnp.sqrt(v + EPS) * gam_ref[...] + bet_ref[...]
    an = jnp.where(an >= 0, an, 0.01 * an)
    out_ref[...] = jnp.dot(an, w2_ref[...],
                           preferred_element_type=jnp.float32) + b2_ref[...]


def _head(gx, g1wT, g1b, gbng, gbnb, g2wT, g2b):
    B, D = gx.shape
    full = lambda s: pl.BlockSpec(s, lambda: (0,) * len(s))
    return pl.pallas_call(
        _head_body,
        in_specs=[full((B, D)), full((D, D)), full((1, D)), full((1, D)),
                  full((1, D)), full((D, D)), full((1, D))],
        out_specs=full((B, D)),
        out_shape=jax.ShapeDtypeStruct((B, D), jnp.float32),
    )(gx, g1wT, g1b, gbng, gbnb, g2wT, g2b)


def _edge_level(x, wT, gam_a, bet_a, w2T, gam_b, bet_b):
    B, N, Cp = x.shape
    idxg = _knn(x)
    Gx = _gather_rows(x.reshape(B * N, Cp),
                      idxg.reshape(B * KNB * N)).reshape(B, KNB, N, Cp)
    Y, st1 = _edge_conv1(Gx, x, wT)
    mn1, sd1 = _xla_stats(Y)
    ga = gam_a.reshape(1, -1)
    ba = bet_a.reshape(1, -1)
    if w2T is None:
        return _bn_lrelu_max(Y, mn1, sd1, ga, ba)
    Z, st2 = _bn_lrelu_conv(Y, mn1, sd1, ga, ba, w2T)
    mn2, sd2 = _xla_stats(Z)
    return _bn_lrelu_max(Z, mn2, sd2, gam_b.reshape(1, -1),
                         bet_b.reshape(1, -1))


def kernel(x, c1w, bn1g, bn1b, c2w, bn2g, bn2b, c3w, bn3g, bn3b, c4w, bn4g,
           bn4b, c5w, bn5g, bn5b, c6w, bn6g, bn6b, g1w, g1b, gbng, gbnb,
           g2w, g2b):
    B, N, C = x.shape
    CP = 16
    xp = jnp.pad(x, ((0, 0), (0, 0), (0, CP - C)))
    x1 = _edge_level(xp, jnp.asarray(c1w.T), bn1g, bn1b,
                     jnp.asarray(c2w.T), bn2g, bn2b)
    x2 = _edge_level(x1, jnp.asarray(c3w.T), bn3g, bn3b,
                     jnp.asarray(c4w.T), bn4g, bn4b)
    x3 = _edge_level(x2, jnp.asarray(c5w.T), bn5g, bn5b, None, None, None)
    Y, st6 = _conv6(x1, x2, x3, jnp.asarray(c6w.T))
    Yt6 = jnp.transpose(Y, (0, 2, 1))
    mn6 = jnp.mean(Yt6, axis=(0, 2)).reshape(1, -1)
    sd6 = jnp.sqrt(jnp.var(Yt6, axis=(0, 2)) + EPS).reshape(1, -1)
    local_x, gx = _bn_lrelu_gmax(Y, mn6, sd6, bn6g.reshape(1, -1),
                                 bn6b.reshape(1, -1))
    gx = gx.reshape(B, -1)
    g = _head(gx, jnp.asarray(g1w.T), g1b.reshape(1, -1),
              gbng.reshape(1, -1), gbnb.reshape(1, -1),
              jnp.asarray(g2w.T), g2b.reshape(1, -1))
    return g, local_x
